# Initial kernel scaffold; baseline (speedup 1.0000x reference)
import jax
import jax.numpy as jnp
from jax.experimental import pallas as pl


def kernel(num_nodes, edge_index, edge_attr, x, node_mask, Wm, bm, We1, be1, W1a, b1a, W1b, b1b, We2, be2, W2a, b2a, W2b, b2b, Wf, bf):
    raise NotImplementedError("write your pallas kernel here")


# speedup vs baseline: 3.9000x; 3.9000x over previous
import functools

import jax
import jax.numpy as jnp
from jax import lax
from jax.experimental import pallas as pl
from jax.experimental.pallas import tpu as pltpu
from jax.experimental.pallas import tpu_sc as plsc

N = 1024
E = 16384
D = 32
DE = 16
K = 8
TEMP = 1.0
GEN_EPS = 1e-7
SENT = -3.0e38
MSENT = 2 ** 30

NW = 32


def _sc_mesh():
    return plsc.VectorSubcoreMesh(core_axis_name="c", subcore_axis_name="s",
                                  num_cores=2, num_subcores=16)


def _escore_body(ea_ref, wm_ref, out_ref):
    out_ref[...] = jnp.dot(ea_ref[...], wm_ref[2 * D:, :],
                           preferred_element_type=jnp.float32)


def _escore(edge_attr, Wm):
    return pl.pallas_call(
        _escore_body,
        out_shape=jax.ShapeDtypeStruct((E, 1), jnp.float32),
    )(edge_attr, Wm)


def _topk_body(esc_ref, g_ref, x_ref, wm_ref, bm_ref, topi_ref):
    x = x_ref[...]
    wm = wm_ref[...]
    a = jnp.dot(x, wm[:D, :], preferred_element_type=jnp.float32)
    b = lax.dot_general(wm[D:2 * D, :], x, (((0,), (1,)), ((), ())),
                        preferred_element_type=jnp.float32)
    esc = esc_ref[...]
    valid = esc > (SENT * 0.5)
    score = jnp.where(valid, esc + a + b + bm_ref[0, 0], -1e9)
    m = jnp.max(score, axis=1, keepdims=True)
    ez = jnp.exp(score - m)
    z = ez / jnp.sum(ez, axis=1, keepdims=True)
    w = z + g_ref[...]
    cols = lax.broadcasted_iota(jnp.int32, (N, N), 1)
    picks = []
    for _ in range(K):
        mx = jnp.max(w, axis=1, keepdims=True)
        am = jnp.min(jnp.where(w == mx, cols, N), axis=1, keepdims=True)
        picks.append(am)
        w = jnp.where(cols == am, -1e9, w)
    topi_ref[...] = jnp.concatenate(picks, axis=1)


def _topk(escmap, g, x, Wm, bm):
    return pl.pallas_call(
        _topk_body,
        out_shape=jax.ShapeDtypeStruct((N, K), jnp.int32),
    )(escmap, g, x, Wm, bm.reshape(1, 1))


def _gather_kernel_body(mmap_hbm, topi_hbm, eaf_hbm, out_hbm,
                        tloc, cellbuf, mres, eaiv, elbuf, earflat, sem):
    wid = lax.axis_index("s") * 2 + lax.axis_index("c")
    base = wid * (N * K // NW)
    pltpu.sync_copy(topi_hbm.at[pl.ds(base, 256)], tloc)
    iota = lax.iota(jnp.int32, 16)
    for c in range(16):
        tv = tloc[pl.ds(c * 16, 16)]
        rows = (base + c * 16 + iota) >> 3
        cell = rows * N + tv
        cellbuf[c // 8, pl.ds((c % 8) * 16, 16)] = cell
    for j in range(2):
        pltpu.async_copy(mmap_hbm.at[cellbuf.at[j]],
                         mres.at[pl.ds(j * 128, 128)], sem).wait()
    for c in range(16):
        v = mres[pl.ds(c * 16, 16)]
        v = jnp.where(v >= E, 0, v)
        eaiv[pl.ds(c * 16, 16)] = v
    for g in range(16):
        evec = eaiv[pl.ds(g * 16, 16)]
        for l in range(16):
            m = g * 16 + l
            j, q = (m * DE) // 128, ((m * DE) % 128) // 16
            elbuf[j, pl.ds(q * 16, 16)] = evec[l] * DE + iota
    handles = [pltpu.async_copy(eaf_hbm.at[elbuf.at[j]],
                                earflat.at[pl.ds(j * 128, 128)], sem)
               for j in range(32)]
    for h in handles:
        h.wait()
    pltpu.sync_copy(earflat, out_hbm.at[pl.ds(base * DE, 256 * DE)])


def _gather_new_ea(mmap_flat, topi_flat, ea_flat):
    return pl.kernel(
        _gather_kernel_body,
        out_type=jax.ShapeDtypeStruct((N * K * DE,), jnp.float32),
        mesh=_sc_mesh(),
        scratch_types=[
            pltpu.VMEM((256,), jnp.int32),
            pltpu.VMEM((2, 128), jnp.int32),
            pltpu.VMEM((256,), jnp.int32),
            pltpu.VMEM((256,), jnp.int32),
            pltpu.VMEM((32, 128), jnp.int32),
            pltpu.VMEM((256 * DE,), jnp.float32),
            pltpu.SemaphoreType.DMA,
        ],
    )(mmap_flat, topi_flat, ea_flat)


def _segsum_body(F, vals_hbm, dst_hbm, zero_hbm, out_hbm,
                 acc_sh, dtmp, vflat, elbuf, zflat, sem):
    c = lax.axis_index("c")
    s = lax.axis_index("s")
    slc = N * F // 16
    pltpu.sync_copy(zero_hbm.at[pl.ds(s * slc, slc)], zflat)
    pltpu.sync_copy(zflat, acc_sh.at[pl.ds(s * slc, slc)])
    plsc.subcore_barrier()
    base = c * (N * K // 2) + s * 256
    pltpu.sync_copy(dst_hbm.at[pl.ds(base, 256)], dtmp)
    pltpu.sync_copy(vals_hbm.at[pl.ds(base * F, 256 * F)], vflat)
    iota = lax.iota(jnp.int32, 16)
    nchunk = 256 * F // 128
    for g in range(16):
        dvec = dtmp[pl.ds(g * 16, 16)]
        for l in range(16):
            m = g * 16 + l
            for t in range(F // 16):
                p = m * F + t * 16
                elbuf[p // 128, pl.ds((p % 128) // 16 * 16, 16)] = \
                    dvec[l] * F + t * 16 + iota
    handles = [pltpu.async_copy(vflat.at[pl.ds(j * 128, 128)],
                                acc_sh.at[elbuf.at[j]], sem, add=True)
               for j in range(nchunk)]
    for h in handles:
        h.wait()
    plsc.subcore_barrier()
    pltpu.sync_copy(acc_sh.at[pl.ds(s * slc, slc)],
                    out_hbm.at[pl.ds(c * N * F + s * slc, slc)])


def _segsum(vals_flat, dst_flat, F):
    out = pl.kernel(
        functools.partial(_segsum_body, F),
        out_type=jax.ShapeDtypeStruct((2 * N * F,), jnp.float32),
        mesh=_sc_mesh(),
        scratch_types=[
            pltpu.VMEM_SHARED((N * F,), jnp.float32),
            pltpu.VMEM((256,), jnp.int32),
            pltpu.VMEM((256 * F,), jnp.float32),
            pltpu.VMEM((256 * F // 128, 128), jnp.int32),
            pltpu.VMEM((N * F // 16,), jnp.float32),
            pltpu.SemaphoreType.DMA,
        ],
    )(vals_flat, dst_flat, jnp.zeros((N * F,), jnp.float32))
    return out.reshape(2, N, F)


def _msg1_body(xr_ref, ea_ref, we1_ref, be1_ref, we2_ref, be2_ref,
               exm_ref, prod_ref, ea2_ref):
    ea = ea_ref[...]
    ea1 = jnp.dot(ea, we1_ref[...], preferred_element_type=jnp.float32) \
        + be1_ref[...]
    msg = jnp.maximum(xr_ref[...] + ea1, 0.0) + GEN_EPS
    exm = jnp.exp(msg)
    exm_ref[...] = exm
    prod_ref[...] = exm * msg
    ea2_ref[...] = jnp.dot(ea, we2_ref[...],
                           preferred_element_type=jnp.float32) + be2_ref[...]


def _msg1(x_rep, new_ea, We1, be1, We2, be2):
    H1 = We2.shape[1]
    return pl.pallas_call(
        _msg1_body,
        out_shape=(
            jax.ShapeDtypeStruct((N * K, D), jnp.float32),
            jax.ShapeDtypeStruct((N * K, D), jnp.float32),
            jax.ShapeDtypeStruct((N * K, H1), jnp.float32),
        ),
    )(x_rep, new_ea, We1, be1.reshape(1, -1), We2, be2.reshape(1, -1))


def _msg2_body(hr_ref, ea2_ref, exm_ref, prod_ref):
    msg = jnp.maximum(hr_ref[...] + ea2_ref[...], 0.0) + GEN_EPS
    exm = jnp.exp(msg)
    exm_ref[...] = exm
    prod_ref[...] = exm * msg


def _msg2(h_rep, ea2):
    F = h_rep.shape[1]
    return pl.pallas_call(
        _msg2_body,
        out_shape=(
            jax.ShapeDtypeStruct((N * K, F), jnp.float32),
            jax.ShapeDtypeStruct((N * K, F), jnp.float32),
        ),
    )(h_rep, ea2)


def _mlp_body(final, num_ref, den_ref, x_ref, wa_ref, ba_ref, wb_ref, bb_ref,
              wf_ref, bf_ref, keep_ref, out_ref):
    num = jnp.sum(num_ref[...], axis=0)
    den = jnp.sum(den_ref[...], axis=0)
    agg = num / (den + 1e-16) + x_ref[...]
    h = jnp.maximum(jnp.dot(agg, wa_ref[...],
                            preferred_element_type=jnp.float32) + ba_ref[...],
                    0.0)
    h = jnp.dot(h, wb_ref[...], preferred_element_type=jnp.float32) \
        + bb_ref[...]
    h = jnp.maximum(h, 0.0)
    if final:
        h = jnp.dot(h, wf_ref[...], preferred_element_type=jnp.float32) \
            + bf_ref[...]
        h = h * keep_ref[...]
    out_ref[...] = h


def _mlp(num, den, x, Wa, ba, Wb, bb, Wf=None, bf=None, keep=None):
    final = Wf is not None
    outdim = Wf.shape[1] if final else Wb.shape[1]
    if not final:
        Wf = jnp.zeros((1, 1), jnp.float32)
        bf = jnp.zeros((1,), jnp.float32)
        keep = jnp.zeros((1, 1), jnp.float32)
    return pl.pallas_call(
        functools.partial(_mlp_body, final),
        out_shape=jax.ShapeDtypeStruct((N, outdim), jnp.float32),
    )(num, den, x, Wa, ba.reshape(1, -1), Wb, bb.reshape(1, -1),
      Wf, bf.reshape(1, -1), keep)


def kernel(num_nodes, edge_index, edge_attr, x, node_mask, Wm, bm,
           We1, be1, W1a, b1a, W1b, b1b, We2, be2, W2a, b2a, W2b, b2b,
           Wf, bf):
    src = edge_index[0]
    dst = edge_index[1]

    escore = _escore(edge_attr, Wm)[:, 0]

    escmap = jnp.full((N, N), SENT, jnp.float32).at[src, dst].set(escore)
    mmap = jnp.full((N * N,), MSENT, jnp.int32) \
        .at[src * N + dst].min(jnp.arange(E, dtype=jnp.int32))

    U = jax.random.uniform(jax.random.key(1), (N, N), dtype=jnp.float32)
    g = -jnp.log(-jnp.log(U + 1e-20) + 1e-20)

    topi = _topk(escmap, g, x, Wm, bm)
    topi_flat = topi.reshape(-1)

    new_ea = _gather_new_ea(mmap, topi_flat,
                            edge_attr.reshape(-1)).reshape(N * K, DE)

    x_rep = jnp.repeat(x, K, axis=0)
    exm1, prod1, ea2 = _msg1(x_rep, new_ea, We1, be1, We2, be2)
    den1 = _segsum(exm1.reshape(-1), topi_flat, D)
    num1 = _segsum(prod1.reshape(-1), topi_flat, D)
    h = _mlp(num1, den1, x, W1a, b1a, W1b, b1b)

    h_rep = jnp.repeat(h, K, axis=0)
    exm2, prod2 = _msg2(h_rep, ea2)
    den2 = _segsum(exm2.reshape(-1), topi_flat, DE)
    num2 = _segsum(prod2.reshape(-1), topi_flat, DE)
    keep = ((node_mask != 0)
            & (jnp.arange(N) < num_nodes)).astype(jnp.float32)[:, None]
    out = _mlp(num2, den2, h, W2a, b2a, W2b, b2b, Wf, bf, keep)
    return out


# speedup vs baseline: 10.7176x; 2.7481x over previous
import functools

import jax
import jax.numpy as jnp
from jax import lax
from jax.experimental import pallas as pl
from jax.experimental.pallas import tpu as pltpu
from jax.experimental.pallas import tpu_sc as plsc

N = 1024
E = 16384
D = 32
DE = 16
K = 8
TEMP = 1.0
GEN_EPS = 1e-7
SENT = -3.0e38
MSENT = 2 ** 30

NW = 32


def _sc_mesh():
    return plsc.VectorSubcoreMesh(core_axis_name="c", subcore_axis_name="s",
                                  num_cores=2, num_subcores=16)


def _escore_body(ea_ref, wm_ref, out_ref):
    out_ref[...] = jnp.dot(ea_ref[...], wm_ref[2 * D:, :],
                           preferred_element_type=jnp.float32)


def _escore(edge_attr, Wm):
    return pl.pallas_call(
        _escore_body,
        out_shape=jax.ShapeDtypeStruct((E, 1), jnp.float32),
    )(edge_attr, Wm)


def _topk_body(esc_ref, g_ref, x_ref, wm_ref, bm_ref, topi_ref):
    x = x_ref[...]
    wm = wm_ref[...]
    a = jnp.dot(x, wm[:D, :], preferred_element_type=jnp.float32)
    b = lax.dot_general(wm[D:2 * D, :], x, (((0,), (1,)), ((), ())),
                        preferred_element_type=jnp.float32)
    esc = esc_ref[...]
    valid = esc > (SENT * 0.5)
    score = jnp.where(valid, esc + a + b + bm_ref[0, 0], -1e9)
    m = jnp.max(score, axis=1, keepdims=True)
    ez = jnp.exp(score - m)
    z = ez / jnp.sum(ez, axis=1, keepdims=True)
    w = z + g_ref[...]
    cols = lax.broadcasted_iota(jnp.int32, (N, N), 1)
    picks = []
    for _ in range(K):
        mx = jnp.max(w, axis=1, keepdims=True)
        am = jnp.min(jnp.where(w == mx, cols, N), axis=1, keepdims=True)
        picks.append(am)
        w = jnp.where(cols == am, -1e9, w)
    topi_ref[...] = jnp.concatenate(picks, axis=1)


def _topk(escmap, g, x, Wm, bm):
    return pl.pallas_call(
        _topk_body,
        out_shape=jax.ShapeDtypeStruct((N, K), jnp.int32),
    )(escmap, g, x, Wm, bm.reshape(1, 1))


def _gather_kernel_body(mmap_hbm, topi_hbm, eaf_hbm, out_hbm,
                        tloc, cellbuf, mres, eaiv, elbuf, earflat, ea0buf,
                        sem):
    wid = lax.axis_index("s") * 2 + lax.axis_index("c")
    base = wid * (N * K // NW)
    pltpu.sync_copy(topi_hbm.at[pl.ds(base, 256)], tloc)
    iota = lax.iota(jnp.int32, 16)
    for c in range(16):
        tv = tloc[pl.ds(c * 16, 16)]
        rows = (base + c * 16 + iota) >> 3
        cell = rows * N + tv
        cellbuf[c // 8, pl.ds((c % 8) * 16, 16)] = cell
    for j in range(2):
        pltpu.async_copy(mmap_hbm.at[cellbuf.at[j]],
                         mres.at[pl.ds(j * 128, 128)], sem).wait()
    pltpu.sync_copy(eaf_hbm.at[pl.ds(0, 16)], ea0buf)
    ea0 = ea0buf[...]
    for c in range(16):
        eaiv[pl.ds(c * 16, 16)] = mres[pl.ds(c * 16, 16)]
    for g in range(16):
        evec = eaiv[pl.ds(g * 16, 16)]
        for l in range(16):
            m = g * 16 + l
            j, q = (m * DE) // 128, ((m * DE) % 128) // 16
            ei = evec[l]
            src_row = jnp.where(ei >= E, m, ei)
            elbuf[j, pl.ds(q * 16, 16)] = src_row * DE + iota
    handles = [pltpu.async_copy(eaf_hbm.at[elbuf.at[j]],
                                earflat.at[pl.ds(j * 128, 128)], sem)
               for j in range(32)]
    for h in handles:
        h.wait()
    for g in range(16):
        evec = eaiv[pl.ds(g * 16, 16)]
        for l in range(16):
            m = g * 16 + l
            rv = earflat[pl.ds(m * DE, 16)]
            earflat[pl.ds(m * DE, 16)] = jnp.where(evec[l] >= E, ea0, rv)
    pltpu.sync_copy(earflat, out_hbm.at[pl.ds(base * DE, 256 * DE)])


def _gather_new_ea(mmap_flat, topi_flat, ea_flat):
    return pl.kernel(
        _gather_kernel_body,
        out_type=jax.ShapeDtypeStruct((N * K * DE,), jnp.float32),
        mesh=_sc_mesh(),
        scratch_types=[
            pltpu.VMEM((256,), jnp.int32),
            pltpu.VMEM((2, 128), jnp.int32),
            pltpu.VMEM((256,), jnp.int32),
            pltpu.VMEM((256,), jnp.int32),
            pltpu.VMEM((32, 128), jnp.int32),
            pltpu.VMEM((256 * DE,), jnp.float32),
            pltpu.VMEM((16,), jnp.float32),
            pltpu.SemaphoreType.DMA,
        ],
    )(mmap_flat, topi_flat, ea_flat)


def _segsum_body(F, vals_hbm, dst_hbm, zero_hbm, out_hbm,
                 acc_sh, dtmp, vflat, elbuf, zflat, sem):
    c = lax.axis_index("c")
    s = lax.axis_index("s")
    slc = N * F // 16
    pltpu.sync_copy(zero_hbm.at[pl.ds(s * slc, slc)], zflat)
    pltpu.sync_copy(zflat, acc_sh.at[pl.ds(s * slc, slc)])
    plsc.subcore_barrier()
    base = c * (N * K // 2) + s * 256
    pltpu.sync_copy(dst_hbm.at[pl.ds(base, 256)], dtmp)
    pltpu.sync_copy(vals_hbm.at[pl.ds(base * F, 256 * F)], vflat)
    iota = lax.iota(jnp.int32, 16)
    nchunk = 256 * F // 128
    for g in range(16):
        dvec = dtmp[pl.ds(g * 16, 16)]
        for l in range(16):
            m = g * 16 + l
            for t in range(F // 16):
                p = m * F + t * 16
                elbuf[p // 128, pl.ds((p % 128) // 16 * 16, 16)] = \
                    dvec[l] * F + t * 16 + iota
    handles = [pltpu.async_copy(vflat.at[pl.ds(j * 128, 128)],
                                acc_sh.at[elbuf.at[j]], sem, add=True)
               for j in range(nchunk)]
    for h in handles:
        h.wait()
    plsc.subcore_barrier()
    pltpu.sync_copy(acc_sh.at[pl.ds(s * slc, slc)],
                    out_hbm.at[pl.ds(c * N * F + s * slc, slc)])


def _segsum(vals_flat, dst_flat, F):
    out = pl.kernel(
        functools.partial(_segsum_body, F),
        out_type=jax.ShapeDtypeStruct((2 * N * F,), jnp.float32),
        mesh=_sc_mesh(),
        scratch_types=[
            pltpu.VMEM_SHARED((N * F,), jnp.float32),
            pltpu.VMEM((256,), jnp.int32),
            pltpu.VMEM((256 * F,), jnp.float32),
            pltpu.VMEM((256 * F // 128, 128), jnp.int32),
            pltpu.VMEM((N * F // 16,), jnp.float32),
            pltpu.SemaphoreType.DMA,
        ],
    )(vals_flat, dst_flat, jnp.zeros((N * F,), jnp.float32))
    return out.reshape(2, N, F)


def _msg1_body(xr_ref, ea_ref, we1_ref, be1_ref, we2_ref, be2_ref,
               exm_ref, prod_ref, ea2_ref):
    ea = ea_ref[...]
    ea1 = jnp.dot(ea, we1_ref[...], preferred_element_type=jnp.float32) \
        + be1_ref[...]
    msg = jnp.maximum(xr_ref[...] + ea1, 0.0) + GEN_EPS
    exm = jnp.exp(msg)
    exm_ref[...] = exm
    prod_ref[...] = exm * msg
    ea2_ref[...] = jnp.dot(ea, we2_ref[...],
                           preferred_element_type=jnp.float32) + be2_ref[...]


def _msg1(x_rep, new_ea, We1, be1, We2, be2):
    H1 = We2.shape[1]
    return pl.pallas_call(
        _msg1_body,
        out_shape=(
            jax.ShapeDtypeStruct((N * K, D), jnp.float32),
            jax.ShapeDtypeStruct((N * K, D), jnp.float32),
            jax.ShapeDtypeStruct((N * K, H1), jnp.float32),
        ),
    )(x_rep, new_ea, We1, be1.reshape(1, -1), We2, be2.reshape(1, -1))


def _msg2_body(hr_ref, ea2_ref, exm_ref, prod_ref):
    msg = jnp.maximum(hr_ref[...] + ea2_ref[...], 0.0) + GEN_EPS
    exm = jnp.exp(msg)
    exm_ref[...] = exm
    prod_ref[...] = exm * msg


def _msg2(h_rep, ea2):
    F = h_rep.shape[1]
    return pl.pallas_call(
        _msg2_body,
        out_shape=(
            jax.ShapeDtypeStruct((N * K, F), jnp.float32),
            jax.ShapeDtypeStruct((N * K, F), jnp.float32),
        ),
    )(h_rep, ea2)


def _mlp_body(final, num_ref, den_ref, x_ref, wa_ref, ba_ref, wb_ref, bb_ref,
              wf_ref, bf_ref, keep_ref, out_ref):
    num = jnp.sum(num_ref[...], axis=0)
    den = jnp.sum(den_ref[...], axis=0)
    agg = num / (den + 1e-16) + x_ref[...]
    h = jnp.maximum(jnp.dot(agg, wa_ref[...],
                            preferred_element_type=jnp.float32) + ba_ref[...],
                    0.0)
    h = jnp.dot(h, wb_ref[...], preferred_element_type=jnp.float32) \
        + bb_ref[...]
    h = jnp.maximum(h, 0.0)
    if final:
        h = jnp.dot(h, wf_ref[...], preferred_element_type=jnp.float32) \
            + bf_ref[...]
        h = h * keep_ref[...]
    out_ref[...] = h


def _mlp(num, den, x, Wa, ba, Wb, bb, Wf=None, bf=None, keep=None):
    final = Wf is not None
    outdim = Wf.shape[1] if final else Wb.shape[1]
    if not final:
        Wf = jnp.zeros((1, 1), jnp.float32)
        bf = jnp.zeros((1,), jnp.float32)
        keep = jnp.zeros((1, 1), jnp.float32)
    return pl.pallas_call(
        functools.partial(_mlp_body, final),
        out_shape=jax.ShapeDtypeStruct((N, outdim), jnp.float32),
    )(num, den, x, Wa, ba.reshape(1, -1), Wb, bb.reshape(1, -1),
      Wf, bf.reshape(1, -1), keep)


def kernel(num_nodes, edge_index, edge_attr, x, node_mask, Wm, bm,
           We1, be1, W1a, b1a, W1b, b1b, We2, be2, W2a, b2a, W2b, b2b,
           Wf, bf):
    src = edge_index[0]
    dst = edge_index[1]

    escore = _escore(edge_attr, Wm)[:, 0]

    escmap = jnp.full((N, N), SENT, jnp.float32).at[src, dst].set(escore)
    mmap = jnp.full((N * N,), MSENT, jnp.int32) \
        .at[src * N + dst].min(jnp.arange(E, dtype=jnp.int32))

    U = jax.random.uniform(jax.random.key(1), (N, N), dtype=jnp.float32)
    g = -jnp.log(-jnp.log(U + 1e-20) + 1e-20)

    topi = _topk(escmap, g, x, Wm, bm)
    topi_flat = topi.reshape(-1)

    new_ea = _gather_new_ea(mmap, topi_flat,
                            edge_attr.reshape(-1)).reshape(N * K, DE)

    x_rep = jnp.repeat(x, K, axis=0)
    exm1, prod1, ea2 = _msg1(x_rep, new_ea, We1, be1, We2, be2)
    den1 = _segsum(exm1.reshape(-1), topi_flat, D)
    num1 = _segsum(prod1.reshape(-1), topi_flat, D)
    h = _mlp(num1, den1, x, W1a, b1a, W1b, b1b)

    h_rep = jnp.repeat(h, K, axis=0)
    exm2, prod2 = _msg2(h_rep, ea2)
    den2 = _segsum(exm2.reshape(-1), topi_flat, DE)
    num2 = _segsum(prod2.reshape(-1), topi_flat, DE)
    keep = ((node_mask != 0)
            & (jnp.arange(N) < num_nodes)).astype(jnp.float32)[:, None]
    out = _mlp(num2, den2, h, W2a, b2a, W2b, b2b, Wf, bf, keep)
    return out


# speedup vs baseline: 10.7839x; 1.0062x over previous
import functools

import jax
import jax.numpy as jnp
from jax import lax
from jax.experimental import pallas as pl
from jax.experimental.pallas import tpu as pltpu
from jax.experimental.pallas import tpu_sc as plsc

N = 1024
E = 16384
D = 32
DE = 16
K = 8
TEMP = 1.0
GEN_EPS = 1e-7
SENT = -3.0e38
MSENT = 2 ** 30

NW = 32


def _sc_mesh():
    return plsc.VectorSubcoreMesh(core_axis_name="c", subcore_axis_name="s",
                                  num_cores=2, num_subcores=16)


def _escore_body(ea_ref, wm_ref, out_ref):
    out_ref[...] = jnp.dot(ea_ref[...], wm_ref[2 * D:, :],
                           preferred_element_type=jnp.float32)


def _escore(edge_attr, Wm):
    return pl.pallas_call(
        _escore_body,
        out_shape=jax.ShapeDtypeStruct((E, 1), jnp.float32),
    )(edge_attr, Wm)


def _buildmaps_body(src_hbm, dst_hbm, esc_hbm, scoreT_hbm, mmap_hbm,
                    sv_, dv_, ev_, sblk, mblk):
    wid = lax.axis_index("s") * 2 + lax.axis_index("c")
    j0 = wid * 32
    pltpu.sync_copy(src_hbm.at[pl.ds(0, E)], sv_)
    pltpu.sync_copy(dst_hbm.at[pl.ds(0, E)], dv_)
    pltpu.sync_copy(esc_hbm.at[pl.ds(0, E)], ev_)

    def init_body(i, carry):
        sentv = jnp.full((16,), SENT, jnp.float32)
        msentv = jnp.full((16,), MSENT, jnp.int32)
        for q in range(8):
            sblk[i, pl.ds(q * 16, 16)] = sentv
            mblk[i, pl.ds(q * 16, 16)] = msentv
        return carry

    lax.fori_loop(0, 256, init_body, 0)

    def chunk_body(ci, carry):
        iota = lax.iota(jnp.int32, 16)
        s16 = sv_[pl.ds(ci * 16, 16)]
        d16 = dv_[pl.ds(ci * 16, 16)]
        e16 = ev_[pl.ds(ci * 16, 16)]
        jl = d16 - j0
        inr = jl.astype(jnp.uint32) < 32
        cv = jnp.where(inr, 1, 0)
        gdn = lax.GatherDimensionNumbers(offset_dims=(),
                                         collapsed_slice_dims=(0,),
                                         start_index_map=(0,))
        for sft in (8, 4, 2, 1):
            cv = cv + lax.gather(
                cv, ((iota + sft) & 15)[:, None], gdn, slice_sizes=(1,),
                mode=lax.GatherScatterMode.PROMISE_IN_BOUNDS)
        cnt = cv[0]
        eid = ci * 16 + iota
        idx = jnp.where(inr, jl * N + s16, 0)
        row = idx >> 7
        col = idx & 127

        @pl.when(cnt == 1)
        def _fast():
            plsc.store_scatter(sblk, [row, col], e16, mask=inr)
            cur = plsc.load_gather(mblk, [row, col], mask=inr)
            plsc.store_scatter(mblk, [row, col], jnp.minimum(cur, eid),
                               mask=inr)

        @pl.when(cnt > 1)
        def _slow():
            inri = jnp.where(inr, 1, 0)
            for l in range(16):
                @pl.when(inri[l] != 0)
                def _lane():
                    lm = iota == l
                    plsc.store_scatter(sblk, [row, col], e16, mask=lm)
                    cur = plsc.load_gather(mblk, [row, col], mask=lm)
                    plsc.store_scatter(mblk, [row, col],
                                       jnp.minimum(cur, eid), mask=lm)

        return carry

    lax.fori_loop(0, E // 16, chunk_body, 0)
    pltpu.sync_copy(sblk, scoreT_hbm.at[wid])
    pltpu.sync_copy(mblk, mmap_hbm.at[wid])


def _buildmaps(src, dst, escore):
    return pl.kernel(
        _buildmaps_body,
        out_type=(
            jax.ShapeDtypeStruct((NW, 256, 128), jnp.float32),
            jax.ShapeDtypeStruct((NW, 256, 128), jnp.int32),
        ),
        mesh=_sc_mesh(),
        compiler_params=pltpu.CompilerParams(needs_layout_passes=False),
        scratch_types=[
            pltpu.VMEM((E,), jnp.int32),
            pltpu.VMEM((E,), jnp.int32),
            pltpu.VMEM((E,), jnp.float32),
            pltpu.VMEM((256, 128), jnp.float32),
            pltpu.VMEM((256, 128), jnp.int32),
        ],
    )(src, dst, escore)


def _topk_body(esc_ref, g_ref, x_ref, wm_ref, bm_ref, topi_ref):
    x = x_ref[...]
    wm = wm_ref[...]
    a = lax.dot_general(wm[:D, :], x, (((0,), (1,)), ((), ())),
                        preferred_element_type=jnp.float32)
    b = jnp.dot(x, wm[D:2 * D, :], preferred_element_type=jnp.float32)
    esc = esc_ref[...]
    valid = esc > (SENT * 0.5)
    score = jnp.where(valid, esc + a + b + bm_ref[0, 0], -1e9)
    m = jnp.max(score, axis=0, keepdims=True)
    ez = jnp.exp(score - m)
    z = ez / jnp.sum(ez, axis=0, keepdims=True)
    w = z + g_ref[...]
    rows = lax.broadcasted_iota(jnp.int32, (N, N), 0)
    picks = []
    for _ in range(K):
        mx = jnp.max(w, axis=0, keepdims=True)
        am = jnp.min(jnp.where(w == mx, rows, N), axis=0, keepdims=True)
        picks.append(am)
        w = jnp.where(rows == am, -1e9, w)
    topi_ref[...] = jnp.concatenate(picks, axis=0)


def _topk(scoreT, gT, x, Wm, bm):
    return pl.pallas_call(
        _topk_body,
        out_shape=jax.ShapeDtypeStruct((K, N), jnp.int32),
    )(scoreT, gT, x, Wm, bm.reshape(1, 1))


def _gather_kernel_body(mmap_hbm, topi_hbm, eaf_hbm, out_hbm,
                        tloc, cellbuf, mres, eaiv, elbuf, earflat, ea0buf,
                        sem):
    wid = lax.axis_index("s") * 2 + lax.axis_index("c")
    base = wid * (N * K // NW)
    pltpu.sync_copy(topi_hbm.at[pl.ds(base, 256)], tloc)
    iota = lax.iota(jnp.int32, 16)
    for c in range(16):
        tv = tloc[pl.ds(c * 16, 16)]
        rows = (base + c * 16 + iota) >> 3
        cell = tv * N + rows
        cellbuf[c // 8, pl.ds((c % 8) * 16, 16)] = cell
    for j in range(2):
        pltpu.async_copy(mmap_hbm.at[cellbuf.at[j]],
                         mres.at[pl.ds(j * 128, 128)], sem).wait()
    pltpu.sync_copy(eaf_hbm.at[pl.ds(0, 16)], ea0buf)
    ea0 = ea0buf[...]
    for c in range(16):
        eaiv[pl.ds(c * 16, 16)] = mres[pl.ds(c * 16, 16)]
    for g in range(16):
        evec = eaiv[pl.ds(g * 16, 16)]
        for l in range(16):
            m = g * 16 + l
            j, q = (m * DE) // 128, ((m * DE) % 128) // 16
            ei = evec[l]
            src_row = jnp.where(ei >= E, m, ei)
            elbuf[j, pl.ds(q * 16, 16)] = src_row * DE + iota
    handles = [pltpu.async_copy(eaf_hbm.at[elbuf.at[j]],
                                earflat.at[pl.ds(j * 128, 128)], sem)
               for j in range(32)]
    for h in handles:
        h.wait()
    for g in range(16):
        evec = eaiv[pl.ds(g * 16, 16)]
        for l in range(16):
            m = g * 16 + l
            rv = earflat[pl.ds(m * DE, 16)]
            earflat[pl.ds(m * DE, 16)] = jnp.where(evec[l] >= E, ea0, rv)
    pltpu.sync_copy(earflat, out_hbm.at[pl.ds(base * DE, 256 * DE)])


def _gather_new_ea(mmap_flat, topi_flat, ea_flat):
    return pl.kernel(
        _gather_kernel_body,
        out_type=jax.ShapeDtypeStruct((N * K * DE,), jnp.float32),
        mesh=_sc_mesh(),
        scratch_types=[
            pltpu.VMEM((256,), jnp.int32),
            pltpu.VMEM((2, 128), jnp.int32),
            pltpu.VMEM((256,), jnp.int32),
            pltpu.VMEM((256,), jnp.int32),
            pltpu.VMEM((32, 128), jnp.int32),
            pltpu.VMEM((256 * DE,), jnp.float32),
            pltpu.VMEM((16,), jnp.float32),
            pltpu.SemaphoreType.DMA,
        ],
    )(mmap_flat, topi_flat, ea_flat)


def _segsum_body(F, vals_hbm, dst_hbm, zero_hbm, out_hbm,
                 acc_sh, dtmp, vflat, elbuf, zflat, sem):
    c = lax.axis_index("c")
    s = lax.axis_index("s")
    slc = N * F // 16
    pltpu.sync_copy(zero_hbm.at[pl.ds(s * slc, slc)], zflat)
    pltpu.sync_copy(zflat, acc_sh.at[pl.ds(s * slc, slc)])
    plsc.subcore_barrier()
    base = c * (N * K // 2) + s * 256
    pltpu.sync_copy(dst_hbm.at[pl.ds(base, 256)], dtmp)
    pltpu.sync_copy(vals_hbm.at[pl.ds(base * F, 256 * F)], vflat)
    iota = lax.iota(jnp.int32, 16)
    nchunk = 256 * F // 128
    for g in range(16):
        dvec = dtmp[pl.ds(g * 16, 16)]
        for l in range(16):
            m = g * 16 + l
            for t in range(F // 16):
                p = m * F + t * 16
                elbuf[p // 128, pl.ds((p % 128) // 16 * 16, 16)] = \
                    dvec[l] * F + t * 16 + iota
    handles = [pltpu.async_copy(vflat.at[pl.ds(j * 128, 128)],
                                acc_sh.at[elbuf.at[j]], sem, add=True)
               for j in range(nchunk)]
    for h in handles:
        h.wait()
    plsc.subcore_barrier()
    pltpu.sync_copy(acc_sh.at[pl.ds(s * slc, slc)],
                    out_hbm.at[pl.ds(c * N * F + s * slc, slc)])


def _segsum(vals_flat, dst_flat, F):
    out = pl.kernel(
        functools.partial(_segsum_body, F),
        out_type=jax.ShapeDtypeStruct((2 * N * F,), jnp.float32),
        mesh=_sc_mesh(),
        scratch_types=[
            pltpu.VMEM_SHARED((N * F,), jnp.float32),
            pltpu.VMEM((256,), jnp.int32),
            pltpu.VMEM((256 * F,), jnp.float32),
            pltpu.VMEM((256 * F // 128, 128), jnp.int32),
            pltpu.VMEM((N * F // 16,), jnp.float32),
            pltpu.SemaphoreType.DMA,
        ],
    )(vals_flat, dst_flat, jnp.zeros((N * F,), jnp.float32))
    return out.reshape(2, N, F)


def _msg1_body(xr_ref, ea_ref, we1_ref, be1_ref, we2_ref, be2_ref,
               exm_ref, prod_ref, ea2_ref):
    ea = ea_ref[...]
    ea1 = jnp.dot(ea, we1_ref[...], preferred_element_type=jnp.float32) \
        + be1_ref[...]
    msg = jnp.maximum(xr_ref[...] + ea1, 0.0) + GEN_EPS
    exm = jnp.exp(msg)
    exm_ref[...] = exm
    prod_ref[...] = exm * msg
    ea2_ref[...] = jnp.dot(ea, we2_ref[...],
                           preferred_element_type=jnp.float32) + be2_ref[...]


def _msg1(x_rep, new_ea, We1, be1, We2, be2):
    H1 = We2.shape[1]
    return pl.pallas_call(
        _msg1_body,
        out_shape=(
            jax.ShapeDtypeStruct((N * K, D), jnp.float32),
            jax.ShapeDtypeStruct((N * K, D), jnp.float32),
            jax.ShapeDtypeStruct((N * K, H1), jnp.float32),
        ),
    )(x_rep, new_ea, We1, be1.reshape(1, -1), We2, be2.reshape(1, -1))


def _msg2_body(hr_ref, ea2_ref, exm_ref, prod_ref):
    msg = jnp.maximum(hr_ref[...] + ea2_ref[...], 0.0) + GEN_EPS
    exm = jnp.exp(msg)
    exm_ref[...] = exm
    prod_ref[...] = exm * msg


def _msg2(h_rep, ea2):
    F = h_rep.shape[1]
    return pl.pallas_call(
        _msg2_body,
        out_shape=(
            jax.ShapeDtypeStruct((N * K, F), jnp.float32),
            jax.ShapeDtypeStruct((N * K, F), jnp.float32),
        ),
    )(h_rep, ea2)


def _mlp_body(final, num_ref, den_ref, x_ref, wa_ref, ba_ref, wb_ref, bb_ref,
              wf_ref, bf_ref, keep_ref, out_ref):
    num = jnp.sum(num_ref[...], axis=0)
    den = jnp.sum(den_ref[...], axis=0)
    agg = num / (den + 1e-16) + x_ref[...]
    h = jnp.maximum(jnp.dot(agg, wa_ref[...],
                            preferred_element_type=jnp.float32) + ba_ref[...],
                    0.0)
    h = jnp.dot(h, wb_ref[...], preferred_element_type=jnp.float32) \
        + bb_ref[...]
    h = jnp.maximum(h, 0.0)
    if final:
        h = jnp.dot(h, wf_ref[...], preferred_element_type=jnp.float32) \
            + bf_ref[...]
        h = h * keep_ref[...]
    out_ref[...] = h


def _mlp(num, den, x, Wa, ba, Wb, bb, Wf=None, bf=None, keep=None):
    final = Wf is not None
    outdim = Wf.shape[1] if final else Wb.shape[1]
    if not final:
        Wf = jnp.zeros((1, 1), jnp.float32)
        bf = jnp.zeros((1,), jnp.float32)
        keep = jnp.zeros((1, 1), jnp.float32)
    return pl.pallas_call(
        functools.partial(_mlp_body, final),
        out_shape=jax.ShapeDtypeStruct((N, outdim), jnp.float32),
    )(num, den, x, Wa, ba.reshape(1, -1), Wb, bb.reshape(1, -1),
      Wf, bf.reshape(1, -1), keep)


def kernel(num_nodes, edge_index, edge_attr, x, node_mask, Wm, bm,
           We1, be1, W1a, b1a, W1b, b1b, We2, be2, W2a, b2a, W2b, b2b,
           Wf, bf):
    src = edge_index[0]
    dst = edge_index[1]

    escore = _escore(edge_attr, Wm)[:, 0]

    scoreT3, mmap3 = _buildmaps(src, dst, escore)
    scoreT = scoreT3.reshape(N, N)
    mmap = mmap3.reshape(-1)

    U = jax.random.uniform(jax.random.key(1), (N, N), dtype=jnp.float32)
    g = -jnp.log(-jnp.log(U + 1e-20) + 1e-20)

    topi8 = _topk(scoreT, g.T, x, Wm, bm)
    topi_flat = topi8.T.reshape(-1)

    new_ea = _gather_new_ea(mmap, topi_flat,
                            edge_attr.reshape(-1)).reshape(N * K, DE)

    x_rep = jnp.repeat(x, K, axis=0)
    exm1, prod1, ea2 = _msg1(x_rep, new_ea, We1, be1, We2, be2)
    den1 = _segsum(exm1.reshape(-1), topi_flat, D)
    num1 = _segsum(prod1.reshape(-1), topi_flat, D)
    h = _mlp(num1, den1, x, W1a, b1a, W1b, b1b)

    h_rep = jnp.repeat(h, K, axis=0)
    exm2, prod2 = _msg2(h_rep, ea2)
    den2 = _segsum(exm2.reshape(-1), topi_flat, DE)
    num2 = _segsum(prod2.reshape(-1), topi_flat, DE)
    keep = ((node_mask != 0)
            & (jnp.arange(N) < num_nodes)).astype(jnp.float32)[:, None]
    out = _mlp(num2, den2, h, W2a, b2a, W2b, b2b, Wf, bf, keep)
    return out


# speedup vs baseline: 14.7759x; 1.3702x over previous
import functools

import jax
import jax.numpy as jnp
from jax import lax
from jax.experimental import pallas as pl
from jax.experimental.pallas import tpu as pltpu
from jax.experimental.pallas import tpu_sc as plsc

N = 1024
E = 16384
D = 32
DE = 16
K = 8
TEMP = 1.0
GEN_EPS = 1e-7
SENT = -3.0e38
MSENT = 2 ** 30

NW = 32


def _sc_mesh():
    return plsc.VectorSubcoreMesh(core_axis_name="c", subcore_axis_name="s",
                                  num_cores=2, num_subcores=16)


def _escore_body(ea_ref, wm_ref, out_ref):
    out_ref[...] = jnp.dot(ea_ref[...], wm_ref[2 * D:, :],
                           preferred_element_type=jnp.float32)


def _escore(edge_attr, Wm):
    return pl.pallas_call(
        _escore_body,
        out_shape=jax.ShapeDtypeStruct((E, 1), jnp.float32),
    )(edge_attr, Wm)


def _buildmaps_body(src_hbm, dst_hbm, esc_hbm, scoreT_hbm, mmap_hbm,
                    sv_, dv_, ev_, sblk, mblk):
    wid = lax.axis_index("s") * 2 + lax.axis_index("c")
    j0 = wid * 32
    pltpu.sync_copy(src_hbm.at[pl.ds(0, E)], sv_)
    pltpu.sync_copy(dst_hbm.at[pl.ds(0, E)], dv_)
    pltpu.sync_copy(esc_hbm.at[pl.ds(0, E)], ev_)

    def init_body(i, carry):
        sentv = jnp.full((16,), SENT, jnp.float32)
        msentv = jnp.full((16,), MSENT, jnp.int32)
        for q in range(8):
            sblk[i, pl.ds(q * 16, 16)] = sentv
            mblk[i, pl.ds(q * 16, 16)] = msentv
        return carry

    lax.fori_loop(0, 256, init_body, 0)

    def half_chunk(ci):
        iota = lax.iota(jnp.int32, 16)
        s16 = sv_[pl.ds(ci * 16, 16)]
        d16 = dv_[pl.ds(ci * 16, 16)]
        e16 = ev_[pl.ds(ci * 16, 16)]
        jl = d16 - j0
        inr = jl.astype(jnp.uint32) < 32
        cnt = plsc.all_reduce_population_count(inr)[0]
        eid = ci * 16 + iota
        idx = jnp.where(inr, jl * N + s16, 0)
        row = idx >> 7
        col = idx & 127
        plsc.store_scatter(sblk, [row, col], e16, mask=inr)
        cur = plsc.load_gather(mblk, [row, col], mask=inr)
        plsc.store_scatter(mblk, [row, col], jnp.minimum(cur, eid),
                           mask=inr)

        @pl.when(cnt > 1)
        def _slow():
            inri = jnp.where(inr, 1, 0)
            for l in range(16):
                @pl.when(inri[l] != 0)
                def _lane():
                    lm = iota == l
                    plsc.store_scatter(sblk, [row, col], e16, mask=lm)
                    cur = plsc.load_gather(mblk, [row, col], mask=lm)
                    plsc.store_scatter(mblk, [row, col],
                                       jnp.minimum(cur, eid), mask=lm)

    def chunk_body(c2, carry):
        half_chunk(c2 * 2)
        half_chunk(c2 * 2 + 1)
        return carry

    lax.fori_loop(0, E // 32, chunk_body, 0)
    pltpu.sync_copy(sblk, scoreT_hbm.at[wid])
    pltpu.sync_copy(mblk, mmap_hbm.at[wid])


def _buildmaps(src, dst, escore):
    return pl.kernel(
        _buildmaps_body,
        out_type=(
            jax.ShapeDtypeStruct((NW, 256, 128), jnp.float32),
            jax.ShapeDtypeStruct((NW, 256, 128), jnp.int32),
        ),
        mesh=_sc_mesh(),
        compiler_params=pltpu.CompilerParams(needs_layout_passes=False),
        scratch_types=[
            pltpu.VMEM((E,), jnp.int32),
            pltpu.VMEM((E,), jnp.int32),
            pltpu.VMEM((E,), jnp.float32),
            pltpu.VMEM((256, 128), jnp.float32),
            pltpu.VMEM((256, 128), jnp.int32),
        ],
    )(src, dst, escore)


def _topk_body(esc_ref, g_ref, x_ref, wm_ref, bm_ref, topi_ref):
    x = x_ref[...]
    wm = wm_ref[...]
    a = lax.dot_general(wm[:D, :], x, (((0,), (1,)), ((), ())),
                        preferred_element_type=jnp.float32)
    b = jnp.dot(x, wm[D:2 * D, :], preferred_element_type=jnp.float32)
    esc = esc_ref[...]
    valid = esc > (SENT * 0.5)
    score = jnp.where(valid, esc + a + b + bm_ref[0, 0], -1e9)
    m = jnp.max(score, axis=0, keepdims=True)
    ez = jnp.exp(score - m)
    z = ez / jnp.sum(ez, axis=0, keepdims=True)
    w = z + g_ref[...]
    rows = lax.broadcasted_iota(jnp.int32, (N, N), 0)
    picks = []
    for _ in range(K):
        mx = jnp.max(w, axis=0, keepdims=True)
        am = jnp.min(jnp.where(w == mx, rows, N), axis=0, keepdims=True)
        picks.append(am)
        w = jnp.where(rows == am, -1e9, w)
    topi_ref[...] = jnp.concatenate(picks, axis=0)


def _topk(scoreT, gT, x, Wm, bm):
    return pl.pallas_call(
        _topk_body,
        out_shape=jax.ShapeDtypeStruct((K, N), jnp.int32),
    )(scoreT, gT, x, Wm, bm.reshape(1, 1))


def _gather_kernel_body(mmap_hbm, topi_hbm, eaf_hbm, out_hbm,
                        tloc, cellbuf, mres, eaiv, elbuf, earflat, ea0buf,
                        sem):
    wid = lax.axis_index("s") * 2 + lax.axis_index("c")
    base = wid * (N * K // NW)
    pltpu.sync_copy(topi_hbm.at[pl.ds(base, 256)], tloc)
    iota = lax.iota(jnp.int32, 16)
    for c in range(16):
        tv = tloc[pl.ds(c * 16, 16)]
        rows = (base + c * 16 + iota) >> 3
        cell = tv * N + rows
        cellbuf[c // 8, pl.ds((c % 8) * 16, 16)] = cell
    for j in range(2):
        pltpu.async_copy(mmap_hbm.at[cellbuf.at[j]],
                         mres.at[pl.ds(j * 128, 128)], sem).wait()
    pltpu.sync_copy(eaf_hbm.at[pl.ds(0, 16)], ea0buf)
    ea0 = ea0buf[...]
    for c in range(16):
        eaiv[pl.ds(c * 16, 16)] = mres[pl.ds(c * 16, 16)]
    for g in range(16):
        evec = eaiv[pl.ds(g * 16, 16)]
        for l in range(16):
            m = g * 16 + l
            j, q = (m * DE) // 128, ((m * DE) % 128) // 16
            ei = evec[l]
            src_row = jnp.where(ei >= E, m, ei)
            elbuf[j, pl.ds(q * 16, 16)] = src_row * DE + iota
    handles = [pltpu.async_copy(eaf_hbm.at[elbuf.at[j]],
                                earflat.at[pl.ds(j * 128, 128)], sem)
               for j in range(32)]
    for h in handles:
        h.wait()
    for g in range(16):
        evec = eaiv[pl.ds(g * 16, 16)]
        for l in range(16):
            m = g * 16 + l
            rv = earflat[pl.ds(m * DE, 16)]
            earflat[pl.ds(m * DE, 16)] = jnp.where(evec[l] >= E, ea0, rv)
    pltpu.sync_copy(earflat, out_hbm.at[pl.ds(base * DE, 256 * DE)])


def _gather_new_ea(mmap_flat, topi_flat, ea_flat):
    return pl.kernel(
        _gather_kernel_body,
        out_type=jax.ShapeDtypeStruct((N * K * DE,), jnp.float32),
        mesh=_sc_mesh(),
        scratch_types=[
            pltpu.VMEM((256,), jnp.int32),
            pltpu.VMEM((2, 128), jnp.int32),
            pltpu.VMEM((256,), jnp.int32),
            pltpu.VMEM((256,), jnp.int32),
            pltpu.VMEM((32, 128), jnp.int32),
            pltpu.VMEM((256 * DE,), jnp.float32),
            pltpu.VMEM((16,), jnp.float32),
            pltpu.SemaphoreType.DMA,
        ],
    )(mmap_flat, topi_flat, ea_flat)


def _segsum_body(F, vals_hbm, dst_hbm, zero_hbm, out_hbm,
                 acc_sh, dtmp, vflat, elbuf, zflat, sem):
    c = lax.axis_index("c")
    s = lax.axis_index("s")
    slc = N * F // 16
    pltpu.sync_copy(zero_hbm.at[pl.ds(s * slc, slc)], zflat)
    pltpu.sync_copy(zflat, acc_sh.at[pl.ds(s * slc, slc)])
    plsc.subcore_barrier()
    base = c * (N * K // 2) + s * 256
    pltpu.sync_copy(dst_hbm.at[pl.ds(base, 256)], dtmp)
    pltpu.sync_copy(vals_hbm.at[pl.ds(base * F, 256 * F)], vflat)
    iota = lax.iota(jnp.int32, 16)
    nchunk = 256 * F // 128
    for g in range(16):
        dvec = dtmp[pl.ds(g * 16, 16)]
        for l in range(16):
            m = g * 16 + l
            for t in range(F // 16):
                p = m * F + t * 16
                elbuf[p // 128, pl.ds((p % 128) // 16 * 16, 16)] = \
                    dvec[l] * F + t * 16 + iota
    handles = [pltpu.async_copy(vflat.at[pl.ds(j * 128, 128)],
                                acc_sh.at[elbuf.at[j]], sem, add=True)
               for j in range(nchunk)]
    for h in handles:
        h.wait()
    plsc.subcore_barrier()
    pltpu.sync_copy(acc_sh.at[pl.ds(s * slc, slc)],
                    out_hbm.at[pl.ds(c * N * F + s * slc, slc)])


def _segsum(vals_flat, dst_flat, F):
    out = pl.kernel(
        functools.partial(_segsum_body, F),
        out_type=jax.ShapeDtypeStruct((2 * N * F,), jnp.float32),
        mesh=_sc_mesh(),
        scratch_types=[
            pltpu.VMEM_SHARED((N * F,), jnp.float32),
            pltpu.VMEM((256,), jnp.int32),
            pltpu.VMEM((256 * F,), jnp.float32),
            pltpu.VMEM((256 * F // 128, 128), jnp.int32),
            pltpu.VMEM((N * F // 16,), jnp.float32),
            pltpu.SemaphoreType.DMA,
        ],
    )(vals_flat, dst_flat, jnp.zeros((N * F,), jnp.float32))
    return out.reshape(2, N, F)


def _msg1_body(xr_ref, ea_ref, we1_ref, be1_ref, we2_ref, be2_ref,
               exm_ref, prod_ref, ea2_ref):
    ea = ea_ref[...]
    ea1 = jnp.dot(ea, we1_ref[...], preferred_element_type=jnp.float32) \
        + be1_ref[...]
    msg = jnp.maximum(xr_ref[...] + ea1, 0.0) + GEN_EPS
    exm = jnp.exp(msg)
    exm_ref[...] = exm
    prod_ref[...] = exm * msg
    ea2_ref[...] = jnp.dot(ea, we2_ref[...],
                           preferred_element_type=jnp.float32) + be2_ref[...]


def _msg1(x_rep, new_ea, We1, be1, We2, be2):
    H1 = We2.shape[1]
    return pl.pallas_call(
        _msg1_body,
        out_shape=(
            jax.ShapeDtypeStruct((N * K, D), jnp.float32),
            jax.ShapeDtypeStruct((N * K, D), jnp.float32),
            jax.ShapeDtypeStruct((N * K, H1), jnp.float32),
        ),
    )(x_rep, new_ea, We1, be1.reshape(1, -1), We2, be2.reshape(1, -1))


def _msg2_body(hr_ref, ea2_ref, exm_ref, prod_ref):
    msg = jnp.maximum(hr_ref[...] + ea2_ref[...], 0.0) + GEN_EPS
    exm = jnp.exp(msg)
    exm_ref[...] = exm
    prod_ref[...] = exm * msg


def _msg2(h_rep, ea2):
    F = h_rep.shape[1]
    return pl.pallas_call(
        _msg2_body,
        out_shape=(
            jax.ShapeDtypeStruct((N * K, F), jnp.float32),
            jax.ShapeDtypeStruct((N * K, F), jnp.float32),
        ),
    )(h_rep, ea2)


def _mlp_body(final, num_ref, den_ref, x_ref, wa_ref, ba_ref, wb_ref, bb_ref,
              wf_ref, bf_ref, keep_ref, out_ref):
    num = jnp.sum(num_ref[...], axis=0)
    den = jnp.sum(den_ref[...], axis=0)
    agg = num / (den + 1e-16) + x_ref[...]
    h = jnp.maximum(jnp.dot(agg, wa_ref[...],
                            preferred_element_type=jnp.float32) + ba_ref[...],
                    0.0)
    h = jnp.dot(h, wb_ref[...], preferred_element_type=jnp.float32) \
        + bb_ref[...]
    h = jnp.maximum(h, 0.0)
    if final:
        h = jnp.dot(h, wf_ref[...], preferred_element_type=jnp.float32) \
            + bf_ref[...]
        h = h * keep_ref[...]
    out_ref[...] = h


def _mlp(num, den, x, Wa, ba, Wb, bb, Wf=None, bf=None, keep=None):
    final = Wf is not None
    outdim = Wf.shape[1] if final else Wb.shape[1]
    if not final:
        Wf = jnp.zeros((1, 1), jnp.float32)
        bf = jnp.zeros((1,), jnp.float32)
        keep = jnp.zeros((1, 1), jnp.float32)
    return pl.pallas_call(
        functools.partial(_mlp_body, final),
        out_shape=jax.ShapeDtypeStruct((N, outdim), jnp.float32),
    )(num, den, x, Wa, ba.reshape(1, -1), Wb, bb.reshape(1, -1),
      Wf, bf.reshape(1, -1), keep)


def kernel(num_nodes, edge_index, edge_attr, x, node_mask, Wm, bm,
           We1, be1, W1a, b1a, W1b, b1b, We2, be2, W2a, b2a, W2b, b2b,
           Wf, bf):
    src = edge_index[0]
    dst = edge_index[1]

    escore = _escore(edge_attr, Wm)[:, 0]

    scoreT3, mmap3 = _buildmaps(src, dst, escore)
    scoreT = scoreT3.reshape(N, N)
    mmap = mmap3.reshape(-1)

    U = jax.random.uniform(jax.random.key(1), (N, N), dtype=jnp.float32)
    g = -jnp.log(-jnp.log(U + 1e-20) + 1e-20)

    topi8 = _topk(scoreT, g.T, x, Wm, bm)
    topi_flat = topi8.T.reshape(-1)

    new_ea = _gather_new_ea(mmap, topi_flat,
                            edge_attr.reshape(-1)).reshape(N * K, DE)

    x_rep = jnp.repeat(x, K, axis=0)
    exm1, prod1, ea2 = _msg1(x_rep, new_ea, We1, be1, We2, be2)
    den1 = _segsum(exm1.reshape(-1), topi_flat, D)
    num1 = _segsum(prod1.reshape(-1), topi_flat, D)
    h = _mlp(num1, den1, x, W1a, b1a, W1b, b1b)

    h_rep = jnp.repeat(h, K, axis=0)
    exm2, prod2 = _msg2(h_rep, ea2)
    den2 = _segsum(exm2.reshape(-1), topi_flat, DE)
    num2 = _segsum(prod2.reshape(-1), topi_flat, DE)
    keep = ((node_mask != 0)
            & (jnp.arange(N) < num_nodes)).astype(jnp.float32)[:, None]
    out = _mlp(num2, den2, h, W2a, b2a, W2b, b2b, Wf, bf, keep)
    return out


# speedup vs baseline: 14.8697x; 1.0063x over previous
import functools

import jax
import jax.numpy as jnp
from jax import lax
from jax.experimental import pallas as pl
from jax.experimental.pallas import tpu as pltpu
from jax.experimental.pallas import tpu_sc as plsc

N = 1024
E = 16384
D = 32
DE = 16
K = 8
TEMP = 1.0
GEN_EPS = 1e-7
SENT = -3.0e38
MSENT = 2 ** 30

NW = 32


def _sc_mesh():
    return plsc.VectorSubcoreMesh(core_axis_name="c", subcore_axis_name="s",
                                  num_cores=2, num_subcores=16)


_GDN = lax.GatherDimensionNumbers(offset_dims=(), collapsed_slice_dims=(0,),
                                  start_index_map=(0,))


def _lane_bcast(v, l):
    idx = jnp.full((16, 1), l, jnp.int32)
    return lax.gather(v, idx, _GDN, slice_sizes=(1,),
                      mode=lax.GatherScatterMode.PROMISE_IN_BOUNDS)


def _escore_body(ea_ref, wm_ref, out_ref):
    out_ref[...] = jnp.dot(ea_ref[...], wm_ref[2 * D:, :],
                           preferred_element_type=jnp.float32)


def _escore(edge_attr, Wm):
    return pl.pallas_call(
        _escore_body,
        out_shape=jax.ShapeDtypeStruct((E, 1), jnp.float32),
    )(edge_attr, Wm)


def _buildmaps_body(src_hbm, dst_hbm, esc_hbm, scoreT_hbm, mmap_hbm,
                    sv_, dv_, ev_, sblk, mblk):
    wid = lax.axis_index("s") * 2 + lax.axis_index("c")
    j0 = wid * 32
    pltpu.sync_copy(src_hbm.at[pl.ds(0, E)], sv_)
    pltpu.sync_copy(dst_hbm.at[pl.ds(0, E)], dv_)
    pltpu.sync_copy(esc_hbm.at[pl.ds(0, E)], ev_)

    def init_body(i, carry):
        sentv = jnp.full((16,), SENT, jnp.float32)
        msentv = jnp.full((16,), MSENT, jnp.int32)
        for q in range(8):
            sblk[i, pl.ds(q * 16, 16)] = sentv
            mblk[i, pl.ds(q * 16, 16)] = msentv
        return carry

    lax.fori_loop(0, 256, init_body, 0)

    def half_chunk(ci):
        iota = lax.iota(jnp.int32, 16)
        s16 = sv_[pl.ds(ci * 16, 16)]
        d16 = dv_[pl.ds(ci * 16, 16)]
        e16 = ev_[pl.ds(ci * 16, 16)]
        jl = d16 - j0
        inr = jl.astype(jnp.uint32) < 32
        cnt = plsc.all_reduce_population_count(inr)[0]
        eid = ci * 16 + iota
        idx = jnp.where(inr, jl * N + s16, 0)
        row = idx >> 7
        col = idx & 127
        plsc.store_scatter(sblk, [row, col], e16, mask=inr)
        cur = plsc.load_gather(mblk, [row, col], mask=inr)
        plsc.store_scatter(mblk, [row, col], jnp.minimum(cur, eid),
                           mask=inr)

        @pl.when(cnt > 1)
        def _slow():
            inri = jnp.where(inr, 1, 0)
            for l in range(16):
                @pl.when(inri[l] != 0)
                def _lane():
                    lm = iota == l
                    plsc.store_scatter(sblk, [row, col], e16, mask=lm)
                    cur = plsc.load_gather(mblk, [row, col], mask=lm)
                    plsc.store_scatter(mblk, [row, col],
                                       jnp.minimum(cur, eid), mask=lm)

    def chunk_body(c2, carry):
        half_chunk(c2 * 2)
        half_chunk(c2 * 2 + 1)
        return carry

    lax.fori_loop(0, E // 32, chunk_body, 0)
    pltpu.sync_copy(sblk, scoreT_hbm.at[wid])
    pltpu.sync_copy(mblk, mmap_hbm.at[wid])


def _buildmaps(src, dst, escore):
    return pl.kernel(
        _buildmaps_body,
        out_type=(
            jax.ShapeDtypeStruct((NW, 256, 128), jnp.float32),
            jax.ShapeDtypeStruct((NW, 256, 128), jnp.int32),
        ),
        mesh=_sc_mesh(),
        compiler_params=pltpu.CompilerParams(needs_layout_passes=False),
        scratch_types=[
            pltpu.VMEM((E,), jnp.int32),
            pltpu.VMEM((E,), jnp.int32),
            pltpu.VMEM((E,), jnp.float32),
            pltpu.VMEM((256, 128), jnp.float32),
            pltpu.VMEM((256, 128), jnp.int32),
        ],
    )(src, dst, escore)


def _topk_body(esc_ref, g_ref, x_ref, wm_ref, bm_ref, topi_ref):
    x = x_ref[...]
    wm = wm_ref[...]
    a = lax.dot_general(wm[:D, :], x, (((0,), (1,)), ((), ())),
                        preferred_element_type=jnp.float32)
    b = jnp.dot(x, wm[D:2 * D, :], preferred_element_type=jnp.float32)
    esc = esc_ref[...]
    valid = esc > (SENT * 0.5)
    score = jnp.where(valid, esc + a + b + bm_ref[0, 0], -1e9)
    m = jnp.max(score, axis=0, keepdims=True)
    ez = jnp.exp(score - m)
    z = ez / jnp.sum(ez, axis=0, keepdims=True)
    w = z + g_ref[...]
    rows = lax.broadcasted_iota(jnp.int32, (N, N), 0)
    picks = []
    for _ in range(K):
        am = jnp.argmax(w, axis=0)[None, :]
        picks.append(am)
        w = jnp.where(rows == am, -1e9, w)
    topi_ref[...] = jnp.concatenate(picks, axis=0)


def _topk(scoreT, gT, x, Wm, bm):
    return pl.pallas_call(
        _topk_body,
        out_shape=jax.ShapeDtypeStruct((K, N), jnp.int32),
    )(scoreT, gT, x, Wm, bm.reshape(1, 1))


def _gather_kernel_body(mmap_hbm, topi_hbm, eaf_hbm, out_hbm,
                        tloc, cellbuf, mres, elbuf, earflat, ea0buf,
                        sem):
    wid = lax.axis_index("s") * 2 + lax.axis_index("c")
    base = wid * (N * K // NW)
    pltpu.sync_copy(topi_hbm.at[pl.ds(base, 256)], tloc)
    iota = lax.iota(jnp.int32, 16)
    for c in range(16):
        tv = tloc[pl.ds(c * 16, 16)]
        rows = (base + c * 16 + iota) >> 3
        cell = tv * N + rows
        cellbuf[c // 8, pl.ds((c % 8) * 16, 16)] = cell
    mh = [pltpu.async_copy(mmap_hbm.at[cellbuf.at[j]],
                           mres.at[pl.ds(j * 128, 128)], sem)
          for j in range(2)]
    pltpu.sync_copy(eaf_hbm.at[pl.ds(0, 16)], ea0buf)
    ea0 = ea0buf[...]
    for h in mh:
        h.wait()
    for g in range(16):
        evec = mres[pl.ds(g * 16, 16)]
        for l in range(16):
            m = g * 16 + l
            j, q = (m * DE) // 128, ((m * DE) % 128) // 16
            ei = _lane_bcast(evec, l)
            src_row = jnp.where(ei >= E, m, ei)
            elbuf[j, pl.ds(q * 16, 16)] = src_row * DE + iota
    handles = [pltpu.async_copy(eaf_hbm.at[elbuf.at[j]],
                                earflat.at[pl.ds(j * 128, 128)], sem)
               for j in range(32)]
    for h in handles:
        h.wait()
    for g in range(16):
        evec = mres[pl.ds(g * 16, 16)]
        for l in range(16):
            m = g * 16 + l
            rv = earflat[pl.ds(m * DE, 16)]
            unm = _lane_bcast(evec, l) >= E
            earflat[pl.ds(m * DE, 16)] = jnp.where(unm, ea0, rv)
    pltpu.sync_copy(earflat, out_hbm.at[pl.ds(base * DE, 256 * DE)])


def _gather_new_ea(mmap_flat, topi_flat, ea_flat):
    return pl.kernel(
        _gather_kernel_body,
        out_type=jax.ShapeDtypeStruct((N * K * DE,), jnp.float32),
        mesh=_sc_mesh(),
        compiler_params=pltpu.CompilerParams(needs_layout_passes=False),
        scratch_types=[
            pltpu.VMEM((256,), jnp.int32),
            pltpu.VMEM((2, 128), jnp.int32),
            pltpu.VMEM((256,), jnp.int32),
            pltpu.VMEM((32, 128), jnp.int32),
            pltpu.VMEM((256 * DE,), jnp.float32),
            pltpu.VMEM((16,), jnp.float32),
            pltpu.SemaphoreType.DMA,
        ],
    )(mmap_flat, topi_flat, ea_flat)


def _segsum_body(F, vals_hbm, dst_hbm, zero_hbm, out_hbm,
                 acc_sh, dtmp, vflat, elbuf, zflat, sem):
    c = lax.axis_index("c")
    s = lax.axis_index("s")
    slc = N * F // 16
    pltpu.sync_copy(zero_hbm.at[pl.ds(s * slc, slc)], zflat)
    pltpu.sync_copy(zflat, acc_sh.at[pl.ds(s * slc, slc)])
    plsc.subcore_barrier()
    base = c * (N * K // 2) + s * 256
    pltpu.sync_copy(dst_hbm.at[pl.ds(base, 256)], dtmp)
    pltpu.sync_copy(vals_hbm.at[pl.ds(base * F, 256 * F)], vflat)
    iota = lax.iota(jnp.int32, 16)
    nchunk = 256 * F // 128
    for g in range(16):
        dvec = dtmp[pl.ds(g * 16, 16)]
        for l in range(16):
            m = g * 16 + l
            for t in range(F // 16):
                p = m * F + t * 16
                elbuf[p // 128, pl.ds((p % 128) // 16 * 16, 16)] = \
                    dvec[l] * F + t * 16 + iota
    handles = [pltpu.async_copy(vflat.at[pl.ds(j * 128, 128)],
                                acc_sh.at[elbuf.at[j]], sem, add=True)
               for j in range(nchunk)]
    for h in handles:
        h.wait()
    plsc.subcore_barrier()
    pltpu.sync_copy(acc_sh.at[pl.ds(s * slc, slc)],
                    out_hbm.at[pl.ds(c * N * F + s * slc, slc)])


def _segsum(vals_flat, dst_flat, F):
    out = pl.kernel(
        functools.partial(_segsum_body, F),
        out_type=jax.ShapeDtypeStruct((2 * N * F,), jnp.float32),
        mesh=_sc_mesh(),
        compiler_params=pltpu.CompilerParams(needs_layout_passes=False),
        scratch_types=[
            pltpu.VMEM_SHARED((N * F,), jnp.float32),
            pltpu.VMEM((256,), jnp.int32),
            pltpu.VMEM((256 * F,), jnp.float32),
            pltpu.VMEM((256 * F // 128, 128), jnp.int32),
            pltpu.VMEM((N * F // 16,), jnp.float32),
            pltpu.SemaphoreType.DMA,
        ],
    )(vals_flat, dst_flat, jnp.zeros((N * F,), jnp.float32))
    return out.reshape(2, N, F)


def _msg1_body(xr_ref, ea_ref, we1_ref, be1_ref, we2_ref, be2_ref,
               exm_ref, prod_ref, ea2_ref):
    ea = ea_ref[...]
    ea1 = jnp.dot(ea, we1_ref[...], preferred_element_type=jnp.float32) \
        + be1_ref[...]
    msg = jnp.maximum(xr_ref[...] + ea1, 0.0) + GEN_EPS
    exm = jnp.exp(msg)
    exm_ref[...] = exm
    prod_ref[...] = exm * msg
    ea2_ref[...] = jnp.dot(ea, we2_ref[...],
                           preferred_element_type=jnp.float32) + be2_ref[...]


def _msg1(x_rep, new_ea, We1, be1, We2, be2):
    H1 = We2.shape[1]
    return pl.pallas_call(
        _msg1_body,
        out_shape=(
            jax.ShapeDtypeStruct((N * K, D), jnp.float32),
            jax.ShapeDtypeStruct((N * K, D), jnp.float32),
            jax.ShapeDtypeStruct((N * K, H1), jnp.float32),
        ),
    )(x_rep, new_ea, We1, be1.reshape(1, -1), We2, be2.reshape(1, -1))


def _msg2_body(hr_ref, ea2_ref, exm_ref, prod_ref):
    msg = jnp.maximum(hr_ref[...] + ea2_ref[...], 0.0) + GEN_EPS
    exm = jnp.exp(msg)
    exm_ref[...] = exm
    prod_ref[...] = exm * msg


def _msg2(h_rep, ea2):
    F = h_rep.shape[1]
    return pl.pallas_call(
        _msg2_body,
        out_shape=(
            jax.ShapeDtypeStruct((N * K, F), jnp.float32),
            jax.ShapeDtypeStruct((N * K, F), jnp.float32),
        ),
    )(h_rep, ea2)


def _mlp_body(final, num_ref, den_ref, x_ref, wa_ref, ba_ref, wb_ref, bb_ref,
              wf_ref, bf_ref, keep_ref, out_ref):
    num = jnp.sum(num_ref[...], axis=0)
    den = jnp.sum(den_ref[...], axis=0)
    agg = num / (den + 1e-16) + x_ref[...]
    h = jnp.maximum(jnp.dot(agg, wa_ref[...],
                            preferred_element_type=jnp.float32) + ba_ref[...],
                    0.0)
    h = jnp.dot(h, wb_ref[...], preferred_element_type=jnp.float32) \
        + bb_ref[...]
    h = jnp.maximum(h, 0.0)
    if final:
        h = jnp.dot(h, wf_ref[...], preferred_element_type=jnp.float32) \
            + bf_ref[...]
        h = h * keep_ref[...]
    out_ref[...] = h


def _mlp(num, den, x, Wa, ba, Wb, bb, Wf=None, bf=None, keep=None):
    final = Wf is not None
    outdim = Wf.shape[1] if final else Wb.shape[1]
    if not final:
        Wf = jnp.zeros((1, 1), jnp.float32)
        bf = jnp.zeros((1,), jnp.float32)
        keep = jnp.zeros((1, 1), jnp.float32)
    return pl.pallas_call(
        functools.partial(_mlp_body, final),
        out_shape=jax.ShapeDtypeStruct((N, outdim), jnp.float32),
    )(num, den, x, Wa, ba.reshape(1, -1), Wb, bb.reshape(1, -1),
      Wf, bf.reshape(1, -1), keep)


def kernel(num_nodes, edge_index, edge_attr, x, node_mask, Wm, bm,
           We1, be1, W1a, b1a, W1b, b1b, We2, be2, W2a, b2a, W2b, b2b,
           Wf, bf):
    src = edge_index[0]
    dst = edge_index[1]

    escore = _escore(edge_attr, Wm)[:, 0]

    scoreT3, mmap3 = _buildmaps(src, dst, escore)
    scoreT = scoreT3.reshape(N, N)
    mmap = mmap3.reshape(-1)

    U = jax.random.uniform(jax.random.key(1), (N, N), dtype=jnp.float32)
    g = -jnp.log(-jnp.log(U + 1e-20) + 1e-20)

    topi8 = _topk(scoreT, g.T, x, Wm, bm)
    topi_flat = topi8.T.reshape(-1)

    new_ea = _gather_new_ea(mmap, topi_flat,
                            edge_attr.reshape(-1)).reshape(N * K, DE)

    x_rep = jnp.repeat(x, K, axis=0)
    exm1, prod1, ea2 = _msg1(x_rep, new_ea, We1, be1, We2, be2)
    den1 = _segsum(exm1.reshape(-1), topi_flat, D)
    num1 = _segsum(prod1.reshape(-1), topi_flat, D)
    h = _mlp(num1, den1, x, W1a, b1a, W1b, b1b)

    h_rep = jnp.repeat(h, K, axis=0)
    exm2, prod2 = _msg2(h_rep, ea2)
    den2 = _segsum(exm2.reshape(-1), topi_flat, DE)
    num2 = _segsum(prod2.reshape(-1), topi_flat, DE)
    keep = ((node_mask != 0)
            & (jnp.arange(N) < num_nodes)).astype(jnp.float32)[:, None]
    out = _mlp(num2, den2, h, W2a, b2a, W2b, b2b, Wf, bf, keep)
    return out


# speedup vs baseline: 15.2203x; 1.0236x over previous
import functools

import jax
import jax.numpy as jnp
from jax import lax
from jax.experimental import pallas as pl
from jax.experimental.pallas import tpu as pltpu
from jax.experimental.pallas import tpu_sc as plsc

N = 1024
E = 16384
D = 32
DE = 16
K = 8
TEMP = 1.0
GEN_EPS = 1e-7
SENT = -3.0e38
MSENT = 2 ** 30

NW = 32


def _sc_mesh():
    return plsc.VectorSubcoreMesh(core_axis_name="c", subcore_axis_name="s",
                                  num_cores=2, num_subcores=16)


_GDN = lax.GatherDimensionNumbers(offset_dims=(), collapsed_slice_dims=(0,),
                                  start_index_map=(0,))


def _lane_bcast(v, l):
    idx = jnp.full((16, 1), l, jnp.int32)
    return lax.gather(v, idx, _GDN, slice_sizes=(1,),
                      mode=lax.GatherScatterMode.PROMISE_IN_BOUNDS)


def _escore_body(ea_ref, wm_ref, out_ref):
    out_ref[...] = jnp.dot(ea_ref[...], wm_ref[2 * D:, :],
                           preferred_element_type=jnp.float32)


def _escore(edge_attr, Wm):
    return pl.pallas_call(
        _escore_body,
        out_shape=jax.ShapeDtypeStruct((E, 1), jnp.float32),
    )(edge_attr, Wm)


def _buildmaps_body(src_hbm, dst_hbm, esc_hbm, scoreT_hbm, mmap_hbm,
                    sv_, dv_, ev_, sblk, mblk):
    wid = lax.axis_index("s") * 2 + lax.axis_index("c")
    j0 = wid * 32
    pltpu.sync_copy(src_hbm.at[pl.ds(0, E)], sv_)
    pltpu.sync_copy(dst_hbm.at[pl.ds(0, E)], dv_)
    pltpu.sync_copy(esc_hbm.at[pl.ds(0, E)], ev_)

    def init_body(i, carry):
        sentv = jnp.full((16,), SENT, jnp.float32)
        msentv = jnp.full((16,), MSENT, jnp.int32)
        for q in range(8):
            sblk[i, pl.ds(q * 16, 16)] = sentv
            mblk[i, pl.ds(q * 16, 16)] = msentv
        return carry

    lax.fori_loop(0, 256, init_body, 0)

    def half_chunk(ci):
        iota = lax.iota(jnp.int32, 16)
        s16 = sv_[pl.ds(ci * 16, 16)]
        d16 = dv_[pl.ds(ci * 16, 16)]
        e16 = ev_[pl.ds(ci * 16, 16)]
        jl = d16 - j0
        inr = jl.astype(jnp.uint32) < 32
        cnt = plsc.all_reduce_population_count(inr)[0]
        eid = ci * 16 + iota
        idx = jnp.where(inr, jl * N + s16, 0)
        row = idx >> 7
        col = idx & 127
        plsc.store_scatter(sblk, [row, col], e16, mask=inr)
        cur = plsc.load_gather(mblk, [row, col], mask=inr)
        plsc.store_scatter(mblk, [row, col], jnp.minimum(cur, eid),
                           mask=inr)

        @pl.when(cnt > 1)
        def _slow():
            inri = jnp.where(inr, 1, 0)
            for l in range(16):
                @pl.when(inri[l] != 0)
                def _lane():
                    lm = iota == l
                    plsc.store_scatter(sblk, [row, col], e16, mask=lm)
                    cur = plsc.load_gather(mblk, [row, col], mask=lm)
                    plsc.store_scatter(mblk, [row, col],
                                       jnp.minimum(cur, eid), mask=lm)

    def chunk_body(c2, carry):
        half_chunk(c2 * 2)
        half_chunk(c2 * 2 + 1)
        return carry

    lax.fori_loop(0, E // 32, chunk_body, 0)
    pltpu.sync_copy(sblk, scoreT_hbm.at[wid])
    pltpu.sync_copy(mblk, mmap_hbm.at[wid])


def _buildmaps(src, dst, escore):
    return pl.kernel(
        _buildmaps_body,
        out_type=(
            jax.ShapeDtypeStruct((NW, 256, 128), jnp.float32),
            jax.ShapeDtypeStruct((NW, 256, 128), jnp.int32),
        ),
        mesh=_sc_mesh(),
        compiler_params=pltpu.CompilerParams(needs_layout_passes=False),
        scratch_types=[
            pltpu.VMEM((E,), jnp.int32),
            pltpu.VMEM((E,), jnp.int32),
            pltpu.VMEM((E,), jnp.float32),
            pltpu.VMEM((256, 128), jnp.float32),
            pltpu.VMEM((256, 128), jnp.int32),
        ],
    )(src, dst, escore)


def _topk_body(esc_ref, g_ref, x_ref, wm_ref, bm_ref, topi_ref):
    x = x_ref[...]
    wm = wm_ref[...]
    a = lax.dot_general(wm[:D, :], x, (((0,), (1,)), ((), ())),
                        preferred_element_type=jnp.float32)
    b = jnp.dot(x, wm[D:2 * D, :], preferred_element_type=jnp.float32)
    esc = esc_ref[...]
    valid = esc > (SENT * 0.5)
    score = jnp.where(valid, esc + a + b + bm_ref[0, 0], -1e9)
    m = jnp.max(score, axis=0, keepdims=True)
    ez = jnp.exp(score - m)
    z = ez / jnp.sum(ez, axis=0, keepdims=True)
    w = z + g_ref[...]
    rows = lax.broadcasted_iota(jnp.int32, (N, N), 0)
    picks = []
    for _ in range(K):
        am = jnp.argmax(w, axis=0)[None, :]
        picks.append(am)
        w = jnp.where(rows == am, -1e9, w)
    topi_ref[...] = jnp.concatenate(picks, axis=0)


def _topk(scoreT, gT, x, Wm, bm):
    return pl.pallas_call(
        _topk_body,
        out_shape=jax.ShapeDtypeStruct((K, N), jnp.int32),
    )(scoreT, gT, x, Wm, bm.reshape(1, 1))


def _gather_kernel_body(mmap_hbm, topi_hbm, eaf_hbm, out_hbm,
                        tloc, cellbuf, mres, elbuf, earflat, ea0buf,
                        sem):
    wid = lax.axis_index("s") * 2 + lax.axis_index("c")
    base = wid * (N * K // NW)
    pltpu.sync_copy(topi_hbm.at[pl.ds(base, 256)], tloc)
    iota = lax.iota(jnp.int32, 16)
    for c in range(16):
        tv = tloc[pl.ds(c * 16, 16)]
        rows = (base + c * 16 + iota) >> 3
        cell = tv * N + rows
        cellbuf[c // 8, pl.ds((c % 8) * 16, 16)] = cell
    mh = [pltpu.async_copy(mmap_hbm.at[cellbuf.at[j]],
                           mres.at[pl.ds(j * 128, 128)], sem)
          for j in range(2)]
    pltpu.sync_copy(eaf_hbm.at[pl.ds(0, 16)], ea0buf)
    ea0 = ea0buf[...]
    for h in mh:
        h.wait()
    for g in range(16):
        evec = mres[pl.ds(g * 16, 16)]
        for l in range(16):
            m = g * 16 + l
            j, q = (m * DE) // 128, ((m * DE) % 128) // 16
            ei = _lane_bcast(evec, l)
            src_row = jnp.where(ei >= E, m, ei)
            elbuf[j, pl.ds(q * 16, 16)] = src_row * DE + iota
    handles = [pltpu.async_copy(eaf_hbm.at[elbuf.at[j]],
                                earflat.at[pl.ds(j * 128, 128)], sem)
               for j in range(32)]
    for h in handles:
        h.wait()
    for g in range(16):
        evec = mres[pl.ds(g * 16, 16)]
        for l in range(16):
            m = g * 16 + l
            rv = earflat[pl.ds(m * DE, 16)]
            unm = _lane_bcast(evec, l) >= E
            earflat[pl.ds(m * DE, 16)] = jnp.where(unm, ea0, rv)
    pltpu.sync_copy(earflat, out_hbm.at[pl.ds(base * DE, 256 * DE)])


def _gather_new_ea(mmap_flat, topi_flat, ea_flat):
    return pl.kernel(
        _gather_kernel_body,
        out_type=jax.ShapeDtypeStruct((N * K * DE,), jnp.float32),
        mesh=_sc_mesh(),
        compiler_params=pltpu.CompilerParams(needs_layout_passes=False),
        scratch_types=[
            pltpu.VMEM((256,), jnp.int32),
            pltpu.VMEM((2, 128), jnp.int32),
            pltpu.VMEM((256,), jnp.int32),
            pltpu.VMEM((32, 128), jnp.int32),
            pltpu.VMEM((256 * DE,), jnp.float32),
            pltpu.VMEM((16,), jnp.float32),
            pltpu.SemaphoreType.DMA,
        ],
    )(mmap_flat, topi_flat, ea_flat)


def _segsum_body(F, den_hbm, num_hbm, dst_hbm, zero_hbm,
                 dout_hbm, nout_hbm,
                 acc_d, acc_n, dtmp, vflat_d, vflat_n, elbuf, zflat, sem):
    c = lax.axis_index("c")
    s = lax.axis_index("s")
    slc = N * F // 16
    pltpu.sync_copy(zero_hbm.at[pl.ds(s * slc, slc)], zflat)
    pltpu.sync_copy(zflat, acc_d.at[pl.ds(s * slc, slc)])
    pltpu.sync_copy(zflat, acc_n.at[pl.ds(s * slc, slc)])
    plsc.subcore_barrier()
    base = c * (N * K // 2) + s * 256
    pltpu.sync_copy(dst_hbm.at[pl.ds(base, 256)], dtmp)
    pltpu.sync_copy(den_hbm.at[pl.ds(base * F, 256 * F)], vflat_d)
    pltpu.sync_copy(num_hbm.at[pl.ds(base * F, 256 * F)], vflat_n)
    iota = lax.iota(jnp.int32, 16)
    nchunk = 256 * F // 128
    for g in range(16):
        dvec = dtmp[pl.ds(g * 16, 16)]
        for l in range(16):
            m = g * 16 + l
            dv = _lane_bcast(dvec, l)
            for t in range(F // 16):
                p = m * F + t * 16
                elbuf[p // 128, pl.ds((p % 128) // 16 * 16, 16)] = \
                    dv * F + t * 16 + iota
    handles = [pltpu.async_copy(vflat_d.at[pl.ds(j * 128, 128)],
                                acc_d.at[elbuf.at[j]], sem, add=True)
               for j in range(nchunk)]
    handles += [pltpu.async_copy(vflat_n.at[pl.ds(j * 128, 128)],
                                 acc_n.at[elbuf.at[j]], sem, add=True)
                for j in range(nchunk)]
    for h in handles:
        h.wait()
    plsc.subcore_barrier()
    pltpu.sync_copy(acc_d.at[pl.ds(s * slc, slc)],
                    dout_hbm.at[pl.ds(c * N * F + s * slc, slc)])
    pltpu.sync_copy(acc_n.at[pl.ds(s * slc, slc)],
                    nout_hbm.at[pl.ds(c * N * F + s * slc, slc)])


def _segsum2(den_flat, num_flat, dst_flat, F):
    dout, nout = pl.kernel(
        functools.partial(_segsum_body, F),
        out_type=(jax.ShapeDtypeStruct((2 * N * F,), jnp.float32),
                  jax.ShapeDtypeStruct((2 * N * F,), jnp.float32)),
        mesh=_sc_mesh(),
        compiler_params=pltpu.CompilerParams(needs_layout_passes=False),
        scratch_types=[
            pltpu.VMEM_SHARED((N * F,), jnp.float32),
            pltpu.VMEM_SHARED((N * F,), jnp.float32),
            pltpu.VMEM((256,), jnp.int32),
            pltpu.VMEM((256 * F,), jnp.float32),
            pltpu.VMEM((256 * F,), jnp.float32),
            pltpu.VMEM((256 * F // 128, 128), jnp.int32),
            pltpu.VMEM((N * F // 16,), jnp.float32),
            pltpu.SemaphoreType.DMA,
        ],
    )(den_flat, num_flat, dst_flat, jnp.zeros((N * F,), jnp.float32))
    return dout.reshape(2, N, F), nout.reshape(2, N, F)


def _msg1_body(xr_ref, ea_ref, we1_ref, be1_ref, we2_ref, be2_ref,
               exm_ref, prod_ref, ea2_ref):
    ea = ea_ref[...]
    ea1 = jnp.dot(ea, we1_ref[...], preferred_element_type=jnp.float32) \
        + be1_ref[...]
    msg = jnp.maximum(xr_ref[...] + ea1, 0.0) + GEN_EPS
    exm = jnp.exp(msg)
    exm_ref[...] = exm
    prod_ref[...] = exm * msg
    ea2_ref[...] = jnp.dot(ea, we2_ref[...],
                           preferred_element_type=jnp.float32) + be2_ref[...]


def _msg1(x_rep, new_ea, We1, be1, We2, be2):
    H1 = We2.shape[1]
    return pl.pallas_call(
        _msg1_body,
        out_shape=(
            jax.ShapeDtypeStruct((N * K, D), jnp.float32),
            jax.ShapeDtypeStruct((N * K, D), jnp.float32),
            jax.ShapeDtypeStruct((N * K, H1), jnp.float32),
        ),
    )(x_rep, new_ea, We1, be1.reshape(1, -1), We2, be2.reshape(1, -1))


def _msg2_body(hr_ref, ea2_ref, exm_ref, prod_ref):
    msg = jnp.maximum(hr_ref[...] + ea2_ref[...], 0.0) + GEN_EPS
    exm = jnp.exp(msg)
    exm_ref[...] = exm
    prod_ref[...] = exm * msg


def _msg2(h_rep, ea2):
    F = h_rep.shape[1]
    return pl.pallas_call(
        _msg2_body,
        out_shape=(
            jax.ShapeDtypeStruct((N * K, F), jnp.float32),
            jax.ShapeDtypeStruct((N * K, F), jnp.float32),
        ),
    )(h_rep, ea2)


def _mlp_body(final, num_ref, den_ref, x_ref, wa_ref, ba_ref, wb_ref, bb_ref,
              wf_ref, bf_ref, keep_ref, out_ref):
    num = jnp.sum(num_ref[...], axis=0)
    den = jnp.sum(den_ref[...], axis=0)
    agg = num / (den + 1e-16) + x_ref[...]
    h = jnp.maximum(jnp.dot(agg, wa_ref[...],
                            preferred_element_type=jnp.float32) + ba_ref[...],
                    0.0)
    h = jnp.dot(h, wb_ref[...], preferred_element_type=jnp.float32) \
        + bb_ref[...]
    h = jnp.maximum(h, 0.0)
    if final:
        h = jnp.dot(h, wf_ref[...], preferred_element_type=jnp.float32) \
            + bf_ref[...]
        h = h * keep_ref[...]
    out_ref[...] = h


def _mlp(num, den, x, Wa, ba, Wb, bb, Wf=None, bf=None, keep=None):
    final = Wf is not None
    outdim = Wf.shape[1] if final else Wb.shape[1]
    if not final:
        Wf = jnp.zeros((1, 1), jnp.float32)
        bf = jnp.zeros((1,), jnp.float32)
        keep = jnp.zeros((1, 1), jnp.float32)
    return pl.pallas_call(
        functools.partial(_mlp_body, final),
        out_shape=jax.ShapeDtypeStruct((N, outdim), jnp.float32),
    )(num, den, x, Wa, ba.reshape(1, -1), Wb, bb.reshape(1, -1),
      Wf, bf.reshape(1, -1), keep)


def kernel(num_nodes, edge_index, edge_attr, x, node_mask, Wm, bm,
           We1, be1, W1a, b1a, W1b, b1b, We2, be2, W2a, b2a, W2b, b2b,
           Wf, bf):
    src = edge_index[0]
    dst = edge_index[1]

    escore = _escore(edge_attr, Wm)[:, 0]

    scoreT3, mmap3 = _buildmaps(src, dst, escore)
    scoreT = scoreT3.reshape(N, N)
    mmap = mmap3.reshape(-1)

    U = jax.random.uniform(jax.random.key(1), (N, N), dtype=jnp.float32)
    g = -jnp.log(-jnp.log(U + 1e-20) + 1e-20)

    topi8 = _topk(scoreT, g.T, x, Wm, bm)
    topi_flat = topi8.T.reshape(-1)

    new_ea = _gather_new_ea(mmap, topi_flat,
                            edge_attr.reshape(-1)).reshape(N * K, DE)

    x_rep = jnp.repeat(x, K, axis=0)
    exm1, prod1, ea2 = _msg1(x_rep, new_ea, We1, be1, We2, be2)
    den1, num1 = _segsum2(exm1.reshape(-1), prod1.reshape(-1), topi_flat, D)
    h = _mlp(num1, den1, x, W1a, b1a, W1b, b1b)

    h_rep = jnp.repeat(h, K, axis=0)
    exm2, prod2 = _msg2(h_rep, ea2)
    den2, num2 = _segsum2(exm2.reshape(-1), prod2.reshape(-1), topi_flat, DE)
    keep = ((node_mask != 0)
            & (jnp.arange(N) < num_nodes)).astype(jnp.float32)[:, None]
    out = _mlp(num2, den2, h, W2a, b2a, W2b, b2b, Wf, bf, keep)
    return out
